# Initial kernel scaffold; baseline (speedup 1.0000x reference)
import jax
import jax.numpy as jnp
from jax.experimental import pallas as pl


def kernel(x, edge_index, params):
    raise NotImplementedError("write your pallas kernel here")


# speedup vs baseline: 11.1341x; 11.1341x over previous
import functools

import jax
import jax.numpy as jnp
from jax import lax
from jax.experimental import pallas as pl
from jax.experimental.pallas import tpu as pltpu
from jax.experimental.pallas import tpu_sc as plsc

N = 10000
E = 320000
NC, NS = 2, 16
D = 128
CD = D // NC
NP = 10240
NPT = NP // NS

EPT = E // NS
K = 80
NCH = EPT // K
SUP = 10
NSUP = NCH // SUP

DD = 8
EPTD = E // (NC * NS)
KD = 80
NCHD = EPTD // KD
SUPD = 25
NSUPD = NCHD // SUPD

ROWS_B = 1000
GRID = N // ROWS_B

_SC_PARAMS = pltpu.CompilerParams(use_tc_tiling_on_sc=False)


def _sc_mesh():
    return plsc.VectorSubcoreMesh(core_axis_name="c", subcore_axis_name="s",
                                  num_cores=NC, num_subcores=NS)


@functools.cache
def _make_deg_sc():
    return functools.partial(
        pl.kernel,
        out_type=jax.ShapeDtypeStruct((NC, NP, DD), jnp.float32),
        mesh=_sc_mesh(),
        scratch_types=[
            pltpu.VMEM((SUPD, KD), jnp.int32),
            pltpu.VMEM((KD, DD), jnp.float32),
            pltpu.VMEM_SHARED((NP, DD), jnp.float32),
        ],
        compiler_params=_SC_PARAMS,
    )(_deg_body)


def _deg_sc(dst_deg):
    ones = jnp.ones((KD, DD), jnp.float32)
    zeros = jnp.zeros((NPT, DD), jnp.float32)
    return _make_deg_sc()(dst_deg, ones, zeros)


def _deg_body(dst_hbm, ones_hbm, zeros_hbm, out_hbm, dst_v, ones_v, acc_sh):
    cid = lax.axis_index("c")
    sid = lax.axis_index("s")
    wid = cid * NS + sid
    pltpu.sync_copy(ones_hbm, ones_v)
    pltpu.sync_copy(zeros_hbm, acc_sh.at[pl.ds(sid * NPT, NPT)])
    plsc.subcore_barrier()

    def outer(s, carry):
        pltpu.sync_copy(dst_hbm.at[wid, pl.ds(s * SUPD, SUPD)], dst_v)

        def body(j, c2):
            pltpu.sync_copy(ones_v, acc_sh.at[dst_v.at[j]], add=True)
            return c2

        return lax.fori_loop(0, SUPD, body, carry)

    lax.fori_loop(0, NSUPD, outer, 0)
    plsc.subcore_barrier()
    pltpu.sync_copy(acc_sh.at[pl.ds(sid * NPT, NPT)],
                    out_hbm.at[cid, pl.ds(sid * NPT, NPT)])


@functools.cache
def _make_prop_sc():
    return functools.partial(
        pl.kernel,
        out_type=jax.ShapeDtypeStruct((NC, NP, CD), jnp.float32),
        mesh=_sc_mesh(),
        scratch_types=[
            pltpu.VMEM((SUP, K), jnp.int32),
            pltpu.VMEM((SUP, K), jnp.int32),
            pltpu.VMEM((K, CD), jnp.float32),
            pltpu.VMEM((K, CD), jnp.float32),
            pltpu.VMEM_SHARED((NP, CD), jnp.float32),
            pltpu.SemaphoreType.DMA,
            pltpu.SemaphoreType.DMA,
        ],
        compiler_params=_SC_PARAMS,
    )(_prop_body)


def _prop_sc(table, src2, dst):
    zeros = jnp.zeros((NPT, CD), jnp.float32)
    return _make_prop_sc()(table.reshape(NC * N, CD), src2, dst, zeros)


def _prop_body(table_hbm, src_hbm, dst_hbm, zeros_hbm, out_hbm,
               src_v, dst_v, rows_a, rows_b, acc_sh, sem_a, sem_b):
    cid = lax.axis_index("c")
    sid = lax.axis_index("s")
    pltpu.sync_copy(zeros_hbm, acc_sh.at[pl.ds(sid * NPT, NPT)])
    plsc.subcore_barrier()

    def outer(s, carry):
        pltpu.sync_copy(src_hbm.at[cid, sid, pl.ds(s * SUP, SUP)], src_v)
        pltpu.sync_copy(dst_hbm.at[sid, pl.ds(s * SUP, SUP)], dst_v)
        pltpu.async_copy(table_hbm.at[src_v.at[0]], rows_a, sem_a)

        def pair(jj, c2):
            j0 = 2 * jj
            pltpu.make_async_copy(table_hbm.at[src_v.at[j0]],
                                  rows_a, sem_a).wait()
            pltpu.async_copy(table_hbm.at[src_v.at[j0 + 1]], rows_b, sem_b)
            pltpu.sync_copy(rows_a, acc_sh.at[dst_v.at[j0]], add=True)
            pltpu.make_async_copy(table_hbm.at[src_v.at[j0 + 1]],
                                  rows_b, sem_b).wait()

            @pl.when(jj < SUP // 2 - 1)
            def _start_next():
                pltpu.async_copy(table_hbm.at[src_v.at[j0 + 2]],
                                 rows_a, sem_a)

            pltpu.sync_copy(rows_b, acc_sh.at[dst_v.at[j0 + 1]], add=True)
            return c2

        return lax.fori_loop(0, SUP // 2, pair, carry)

    lax.fori_loop(0, NSUP, outer, 0)
    plsc.subcore_barrier()
    pltpu.sync_copy(acc_sh.at[pl.ds(sid * NPT, NPT)],
                    out_hbm.at[cid, pl.ds(sid * NPT, NPT)])


def _split(v, out_ref):
    out_ref[0] = v[:, :CD]
    out_ref[1] = v[:, CD:]


def _cat(ref):
    return jnp.concatenate([ref[0], ref[1]], axis=-1)


def _first_body(x_ref, w_ref, degp_ref, t_ref, dinv_ref):
    deg = 1.0 + sum(degp_ref[i] for i in range(NC))
    dinv = lax.rsqrt(deg)
    dinv_ref[...] = dinv
    z = jnp.dot(x_ref[...], w_ref[...], preferred_element_type=jnp.float32,
                precision=lax.Precision.HIGHEST)
    _split(z * dinv[:, :1], t_ref)


def _tc_first(x, w1p, degp):
    return pl.pallas_call(
        _first_body,
        grid=(GRID,),
        in_specs=[
            pl.BlockSpec((ROWS_B, 128), lambda i: (i, 0)),
            pl.BlockSpec((128, D), lambda i: (0, 0)),
            pl.BlockSpec((NC, ROWS_B, DD), lambda i: (0, i, 0)),
        ],
        out_specs=[
            pl.BlockSpec((NC, ROWS_B, CD), lambda i: (0, i, 0)),
            pl.BlockSpec((ROWS_B, DD), lambda i: (i, 0)),
        ],
        out_shape=[
            jax.ShapeDtypeStruct((NC, N, CD), jnp.float32),
            jax.ShapeDtypeStruct((N, DD), jnp.float32),
        ],
    )(x, w1p, degp)


def _mid_body(q_ref, t_ref, dinv_ref, w_ref, p_ref, out_ref):
    dinv = dinv_ref[...][:, :1]
    pp = p_ref[...]
    z = (_cat(q_ref) + _cat(t_ref)) * dinv + pp[0:1]
    h = jnp.maximum(z, 0.0)
    h = (h - pp[3:4]) * lax.rsqrt(pp[4:5] + 1e-5) * pp[1:2] + pp[2:3]
    t2 = jnp.dot(h, w_ref[...], preferred_element_type=jnp.float32,
                 precision=lax.Precision.HIGHEST)
    _split(t2 * dinv, out_ref)


def _tc_mid(q, t, dinv, w, p):
    return pl.pallas_call(
        _mid_body,
        grid=(GRID,),
        in_specs=[
            pl.BlockSpec((NC, ROWS_B, CD), lambda i: (0, i, 0)),
            pl.BlockSpec((NC, ROWS_B, CD), lambda i: (0, i, 0)),
            pl.BlockSpec((ROWS_B, DD), lambda i: (i, 0)),
            pl.BlockSpec((D, D), lambda i: (0, 0)),
            pl.BlockSpec((8, D), lambda i: (0, 0)),
        ],
        out_specs=pl.BlockSpec((NC, ROWS_B, CD), lambda i: (0, i, 0)),
        out_shape=jax.ShapeDtypeStruct((NC, N, CD), jnp.float32),
    )(q, t, dinv, w, p)


def _head_body(q_ref, t_ref, dinv_ref, w1_ref, w2_ref, w3_ref, p_ref, out_ref):
    dinv = dinv_ref[...][:, :1]
    pp = p_ref[...]
    z = (_cat(q_ref) + _cat(t_ref)) * dinv + pp[0:1]
    h = jnp.maximum(z, 0.0)
    h = (h - pp[3:4]) * lax.rsqrt(pp[4:5] + 1e-5) * pp[1:2] + pp[2:3]
    u = jnp.dot(h, w1_ref[...], preferred_element_type=jnp.float32,
                precision=lax.Precision.HIGHEST) + pp[5:6]
    u = (u - pp[8:9]) * lax.rsqrt(pp[9:10] + 1e-5) * pp[6:7] + pp[7:8]
    u = jnp.maximum(u, 0.0)
    v = jnp.dot(u, w2_ref[...], preferred_element_type=jnp.float32,
                precision=lax.Precision.HIGHEST) + pp[10:11]
    v = (v - pp[13:14]) * lax.rsqrt(pp[14:15] + 1e-5) * pp[11:12] + pp[12:13]
    v = jnp.maximum(v, 0.0)
    y = jnp.dot(v, w3_ref[...], preferred_element_type=jnp.float32,
                precision=lax.Precision.HIGHEST) + pp[15:16, :8]
    out_ref[...] = y


def _tc_head(q, t, dinv, w1, w2, w3, p):
    return pl.pallas_call(
        _head_body,
        grid=(GRID,),
        in_specs=[
            pl.BlockSpec((NC, ROWS_B, CD), lambda i: (0, i, 0)),
            pl.BlockSpec((NC, ROWS_B, CD), lambda i: (0, i, 0)),
            pl.BlockSpec((ROWS_B, DD), lambda i: (i, 0)),
            pl.BlockSpec((D, D), lambda i: (0, 0)),
            pl.BlockSpec((D, D), lambda i: (0, 0)),
            pl.BlockSpec((D, 8), lambda i: (0, 0)),
            pl.BlockSpec((16, D), lambda i: (0, 0)),
        ],
        out_specs=pl.BlockSpec((ROWS_B, 8), lambda i: (i, 0)),
        out_shape=jax.ShapeDtypeStruct((N, 8), jnp.float32),
    )(q, t, dinv, w1, w2, w3, p)


def _padv(v, val=0.0):
    return jnp.pad(v.astype(jnp.float32), (0, D - v.shape[0]),
                   constant_values=val)


def _padm(w, rows=D, cols=D):
    return jnp.pad(w.astype(jnp.float32),
                   ((0, rows - w.shape[0]), (0, cols - w.shape[1])))


def kernel(x, edge_index, params):
    src = edge_index[0].astype(jnp.int32).reshape(NS, NCH, K)
    src2 = jnp.stack([src, src + N])
    dst = edge_index[1].astype(jnp.int32).reshape(NS, NCH, K)
    dst_deg = edge_index[1].astype(jnp.int32).reshape(NC * NS, NCHD, KD)

    w1p = _padm(params['W1'], 128, D)
    wsp = [_padm(params['Ws'][i]) for i in range(3)]
    one = jnp.ones((0,), jnp.float32)
    zero = jnp.zeros((0,), jnp.float32)
    ident = [_padv(one, 1.0), _padv(zero), _padv(zero),
             _padv(one, 1.0 - 1e-5)]
    bn1 = [_padv(params['bn1_g'], 1.0), _padv(params['bn1_b']),
           _padv(params['bn1_m']), _padv(params['bn1_v'], 1.0 - 1e-5)]
    bn2 = [_padv(params['bn2_g'], 1.0), _padv(params['bn2_b']),
           _padv(params['bn2_m']), _padv(params['bn2_v'], 1.0 - 1e-5)]
    bn3 = [_padv(params['bn3_g'], 1.0), _padv(params['bn3_b']),
           _padv(params['bn3_m']), _padv(params['bn3_v'], 1.0 - 1e-5)]

    def pack(rows, total):
        z = jnp.zeros((total - len(rows), D), jnp.float32)
        return jnp.concatenate([jnp.stack(rows), z], axis=0)

    p_mid0 = pack([_padv(params['b1'])] + ident, 8)
    p_mid1 = pack([_padv(params['bs'][0])] + bn1, 8)
    p_mid2 = pack([_padv(params['bs'][1])] + bn1, 8)
    p_head = jnp.stack(
        [_padv(params['bs'][2])] + bn1 +
        [_padv(params['fc1_b'])] + bn2 +
        [_padv(params['fc2_b'])] + bn3 +
        [jnp.full((D,), params['fc3_b'][0], jnp.float32)])
    wf1 = _padm(params['fc1_W'])
    wf2 = _padm(params['fc2_W'])
    wf3 = _padm(params['fc3_W'], D, 8)

    degp = _deg_sc(dst_deg)
    t0, dinv = _tc_first(x, w1p, degp)

    w_stack = jnp.stack([wsp[0], wsp[1], wsp[2], wsp[2]])
    p_stack = jnp.stack([p_mid0, p_mid1, p_mid2, p_mid2])

    def body(carry, xs):
        _, t, _ = carry
        w, p = xs
        q = _prop_sc(t, src2, dst)
        t_new = _tc_mid(q, t, dinv, w, p)
        return (t, t_new, q), None

    q0 = jnp.zeros((NC, NP, CD), jnp.float32)
    (t3, _, q3), _ = lax.scan(body, (t0, t0, q0), (w_stack, p_stack))
    y = _tc_head(q3, t3, dinv, wf1, wf2, wf3, p_head)
    return y[:, :1]


# speedup vs baseline: 12.3734x; 1.1113x over previous
import functools

import jax
import jax.numpy as jnp
from jax import lax
from jax.experimental import pallas as pl
from jax.experimental.pallas import tpu as pltpu
from jax.experimental.pallas import tpu_sc as plsc

N = 10000
E = 320000
NC, NS = 2, 16
D = 128
CD = D // NC
NP = 10240
NPT = NP // NS

EPT = E // NS
K = 100
NCH = EPT // K
SUP = 20
NSUP = NCH // SUP

DD = 8
EPTD = E // (NC * NS)
KD = 80
NCHD = EPTD // KD
SUPD = 5
NSUPD = NCHD // SUPD

ROWS_B = 1000
GRID = N // ROWS_B

_SC_PARAMS = pltpu.CompilerParams(use_tc_tiling_on_sc=False)


def _sc_mesh():
    return plsc.VectorSubcoreMesh(core_axis_name="c", subcore_axis_name="s",
                                  num_cores=NC, num_subcores=NS)


@functools.cache
def _make_deg_sc():
    return functools.partial(
        pl.kernel,
        out_type=jax.ShapeDtypeStruct((NC, NP, DD), jnp.float32),
        mesh=_sc_mesh(),
        scratch_types=[
            pltpu.VMEM((SUPD, KD), jnp.int32),
            pltpu.VMEM((KD, DD), jnp.float32),
            pltpu.VMEM_SHARED((NP, DD), jnp.float32),
        ],
        compiler_params=_SC_PARAMS,
    )(_deg_body)


def _deg_sc(dst_deg):
    ones = jnp.ones((KD, DD), jnp.float32)
    zeros = jnp.zeros((NPT, DD), jnp.float32)
    return _make_deg_sc()(dst_deg, ones, zeros)


def _deg_body(dst_hbm, ones_hbm, zeros_hbm, out_hbm, dst_v, ones_v, acc_sh):
    cid = lax.axis_index("c")
    sid = lax.axis_index("s")
    wid = cid * NS + sid
    pltpu.sync_copy(ones_hbm, ones_v)
    pltpu.sync_copy(zeros_hbm, acc_sh.at[pl.ds(sid * NPT, NPT)])
    plsc.subcore_barrier()

    def outer(s, carry):
        pltpu.sync_copy(dst_hbm.at[wid, pl.ds(s * SUPD, SUPD)], dst_v)

        def body(j, c2):
            pltpu.sync_copy(ones_v, acc_sh.at[dst_v.at[j]], add=True)
            return c2

        return lax.fori_loop(0, SUPD, body, carry)

    lax.fori_loop(0, NSUPD, outer, 0)
    plsc.subcore_barrier()
    pltpu.sync_copy(acc_sh.at[pl.ds(sid * NPT, NPT)],
                    out_hbm.at[cid, pl.ds(sid * NPT, NPT)])


@functools.cache
def _make_prop_sc():
    return functools.partial(
        pl.kernel,
        out_type=jax.ShapeDtypeStruct((NC, NP, CD), jnp.float32),
        mesh=_sc_mesh(),
        scratch_types=[
            pltpu.VMEM((SUP, K), jnp.int32),
            pltpu.VMEM((SUP, K), jnp.int32),
            pltpu.VMEM((K, CD), jnp.float32),
            pltpu.VMEM((K, CD), jnp.float32),
            pltpu.VMEM_SHARED((NP, CD), jnp.float32),
            pltpu.SemaphoreType.DMA,
            pltpu.SemaphoreType.DMA,
        ],
        compiler_params=_SC_PARAMS,
    )(_prop_body)


def _prop_sc(table, src2, dst):
    zeros = jnp.zeros((NPT, CD), jnp.float32)
    return _make_prop_sc()(table.reshape(NC * N, CD), src2, dst, zeros)


def _prop_body(table_hbm, src_hbm, dst_hbm, zeros_hbm, out_hbm,
               src_v, dst_v, rows_a, rows_b, acc_sh, sem_a, sem_b):
    cid = lax.axis_index("c")
    sid = lax.axis_index("s")
    pltpu.sync_copy(zeros_hbm, acc_sh.at[pl.ds(sid * NPT, NPT)])
    plsc.subcore_barrier()

    def outer(s, carry):
        pltpu.sync_copy(src_hbm.at[cid, sid, pl.ds(s * SUP, SUP)], src_v)
        pltpu.sync_copy(dst_hbm.at[sid, pl.ds(s * SUP, SUP)], dst_v)
        pltpu.async_copy(table_hbm.at[src_v.at[0]], rows_a, sem_a)

        def pair(jj, c2):
            j0 = 2 * jj
            pltpu.make_async_copy(table_hbm.at[src_v.at[j0]],
                                  rows_a, sem_a).wait()
            pltpu.async_copy(table_hbm.at[src_v.at[j0 + 1]], rows_b, sem_b)
            pltpu.sync_copy(rows_a, acc_sh.at[dst_v.at[j0]], add=True)
            pltpu.make_async_copy(table_hbm.at[src_v.at[j0 + 1]],
                                  rows_b, sem_b).wait()

            @pl.when(jj < SUP // 2 - 1)
            def _start_next():
                pltpu.async_copy(table_hbm.at[src_v.at[j0 + 2]],
                                 rows_a, sem_a)

            pltpu.sync_copy(rows_b, acc_sh.at[dst_v.at[j0 + 1]], add=True)
            return c2

        return lax.fori_loop(0, SUP // 2, pair, carry)

    lax.fori_loop(0, NSUP, outer, 0)
    plsc.subcore_barrier()
    pltpu.sync_copy(acc_sh.at[pl.ds(sid * NPT, NPT)],
                    out_hbm.at[cid, pl.ds(sid * NPT, NPT)])


def _split(v, out_ref):
    out_ref[0] = v[:, :CD]
    out_ref[1] = v[:, CD:]


def _cat(ref):
    return jnp.concatenate([ref[0], ref[1]], axis=-1)


def _first_body(x_ref, w_ref, degp_ref, t_ref, dinv_ref):
    deg = 1.0 + sum(degp_ref[i] for i in range(NC))
    dinv = lax.rsqrt(deg)
    dinv_ref[...] = dinv
    z = jnp.dot(x_ref[...], w_ref[...], preferred_element_type=jnp.float32,
                precision=lax.Precision.HIGHEST)
    _split(z * dinv[:, :1], t_ref)


def _tc_first(x, w1p, degp):
    return pl.pallas_call(
        _first_body,
        grid=(GRID,),
        in_specs=[
            pl.BlockSpec((ROWS_B, 128), lambda i: (i, 0)),
            pl.BlockSpec((128, D), lambda i: (0, 0)),
            pl.BlockSpec((NC, ROWS_B, DD), lambda i: (0, i, 0)),
        ],
        out_specs=[
            pl.BlockSpec((NC, ROWS_B, CD), lambda i: (0, i, 0)),
            pl.BlockSpec((ROWS_B, DD), lambda i: (i, 0)),
        ],
        out_shape=[
            jax.ShapeDtypeStruct((NC, N, CD), jnp.float32),
            jax.ShapeDtypeStruct((N, DD), jnp.float32),
        ],
    )(x, w1p, degp)


def _mid_body(q_ref, t_ref, dinv_ref, w_ref, p_ref, out_ref):
    dinv = dinv_ref[...][:, :1]
    pp = p_ref[...]
    z = (_cat(q_ref) + _cat(t_ref)) * dinv + pp[0:1]
    h = jnp.maximum(z, 0.0)
    h = (h - pp[3:4]) * lax.rsqrt(pp[4:5] + 1e-5) * pp[1:2] + pp[2:3]
    t2 = jnp.dot(h, w_ref[...], preferred_element_type=jnp.float32,
                 precision=lax.Precision.HIGHEST)
    _split(t2 * dinv, out_ref)


def _tc_mid(q, t, dinv, w, p):
    return pl.pallas_call(
        _mid_body,
        grid=(GRID,),
        in_specs=[
            pl.BlockSpec((NC, ROWS_B, CD), lambda i: (0, i, 0)),
            pl.BlockSpec((NC, ROWS_B, CD), lambda i: (0, i, 0)),
            pl.BlockSpec((ROWS_B, DD), lambda i: (i, 0)),
            pl.BlockSpec((D, D), lambda i: (0, 0)),
            pl.BlockSpec((8, D), lambda i: (0, 0)),
        ],
        out_specs=pl.BlockSpec((NC, ROWS_B, CD), lambda i: (0, i, 0)),
        out_shape=jax.ShapeDtypeStruct((NC, N, CD), jnp.float32),
    )(q, t, dinv, w, p)


def _head_body(q_ref, t_ref, dinv_ref, w1_ref, w2_ref, w3_ref, p_ref, out_ref):
    dinv = dinv_ref[...][:, :1]
    pp = p_ref[...]
    z = (_cat(q_ref) + _cat(t_ref)) * dinv + pp[0:1]
    h = jnp.maximum(z, 0.0)
    h = (h - pp[3:4]) * lax.rsqrt(pp[4:5] + 1e-5) * pp[1:2] + pp[2:3]
    u = jnp.dot(h, w1_ref[...], preferred_element_type=jnp.float32,
                precision=lax.Precision.HIGHEST) + pp[5:6]
    u = (u - pp[8:9]) * lax.rsqrt(pp[9:10] + 1e-5) * pp[6:7] + pp[7:8]
    u = jnp.maximum(u, 0.0)
    v = jnp.dot(u, w2_ref[...], preferred_element_type=jnp.float32,
                precision=lax.Precision.HIGHEST) + pp[10:11]
    v = (v - pp[13:14]) * lax.rsqrt(pp[14:15] + 1e-5) * pp[11:12] + pp[12:13]
    v = jnp.maximum(v, 0.0)
    y = jnp.dot(v, w3_ref[...], preferred_element_type=jnp.float32,
                precision=lax.Precision.HIGHEST) + pp[15:16, :8]
    out_ref[...] = y


def _tc_head(q, t, dinv, w1, w2, w3, p):
    return pl.pallas_call(
        _head_body,
        grid=(GRID,),
        in_specs=[
            pl.BlockSpec((NC, ROWS_B, CD), lambda i: (0, i, 0)),
            pl.BlockSpec((NC, ROWS_B, CD), lambda i: (0, i, 0)),
            pl.BlockSpec((ROWS_B, DD), lambda i: (i, 0)),
            pl.BlockSpec((D, D), lambda i: (0, 0)),
            pl.BlockSpec((D, D), lambda i: (0, 0)),
            pl.BlockSpec((D, 8), lambda i: (0, 0)),
            pl.BlockSpec((16, D), lambda i: (0, 0)),
        ],
        out_specs=pl.BlockSpec((ROWS_B, 8), lambda i: (i, 0)),
        out_shape=jax.ShapeDtypeStruct((N, 8), jnp.float32),
    )(q, t, dinv, w1, w2, w3, p)


def _padv(v, val=0.0):
    return jnp.pad(v.astype(jnp.float32), (0, D - v.shape[0]),
                   constant_values=val)


def _padm(w, rows=D, cols=D):
    return jnp.pad(w.astype(jnp.float32),
                   ((0, rows - w.shape[0]), (0, cols - w.shape[1])))


def kernel(x, edge_index, params):
    src = edge_index[0].astype(jnp.int32).reshape(NS, NCH, K)
    src2 = jnp.stack([src, src + N])
    dst = edge_index[1].astype(jnp.int32).reshape(NS, NCH, K)
    dst_deg = edge_index[1].astype(jnp.int32).reshape(NC * NS, NCHD, KD)

    w1p = _padm(params['W1'], 128, D)
    wsp = [_padm(params['Ws'][i]) for i in range(3)]
    one = jnp.ones((0,), jnp.float32)
    zero = jnp.zeros((0,), jnp.float32)
    ident = [_padv(one, 1.0), _padv(zero), _padv(zero),
             _padv(one, 1.0 - 1e-5)]
    bn1 = [_padv(params['bn1_g'], 1.0), _padv(params['bn1_b']),
           _padv(params['bn1_m']), _padv(params['bn1_v'], 1.0 - 1e-5)]
    bn2 = [_padv(params['bn2_g'], 1.0), _padv(params['bn2_b']),
           _padv(params['bn2_m']), _padv(params['bn2_v'], 1.0 - 1e-5)]
    bn3 = [_padv(params['bn3_g'], 1.0), _padv(params['bn3_b']),
           _padv(params['bn3_m']), _padv(params['bn3_v'], 1.0 - 1e-5)]

    def pack(rows, total):
        z = jnp.zeros((total - len(rows), D), jnp.float32)
        return jnp.concatenate([jnp.stack(rows), z], axis=0)

    p_mid0 = pack([_padv(params['b1'])] + ident, 8)
    p_mid1 = pack([_padv(params['bs'][0])] + bn1, 8)
    p_mid2 = pack([_padv(params['bs'][1])] + bn1, 8)
    p_head = jnp.stack(
        [_padv(params['bs'][2])] + bn1 +
        [_padv(params['fc1_b'])] + bn2 +
        [_padv(params['fc2_b'])] + bn3 +
        [jnp.full((D,), params['fc3_b'][0], jnp.float32)])
    wf1 = _padm(params['fc1_W'])
    wf2 = _padm(params['fc2_W'])
    wf3 = _padm(params['fc3_W'], D, 8)

    degp = _deg_sc(dst_deg)
    t0, dinv = _tc_first(x, w1p, degp)

    w_stack = jnp.stack([wsp[0], wsp[1], wsp[2], wsp[2]])
    p_stack = jnp.stack([p_mid0, p_mid1, p_mid2, p_mid2])

    def body(carry, xs):
        _, t, _ = carry
        w, p = xs
        q = _prop_sc(t, src2, dst)
        t_new = _tc_mid(q, t, dinv, w, p)
        return (t, t_new, q), None

    q0 = jnp.zeros((NC, NP, CD), jnp.float32)
    (t3, _, q3), _ = lax.scan(body, (t0, t0, q0), (w_stack, p_stack))
    y = _tc_head(q3, t3, dinv, wf1, wf2, wf3, p_head)
    return y[:, :1]
